# Initial kernel scaffold; baseline (speedup 1.0000x reference)
import jax
import jax.numpy as jnp
from jax.experimental import pallas as pl


def kernel(table, left, right):
    raise NotImplementedError("write your pallas kernel here")


# speedup vs baseline: 3.5045x; 3.5045x over previous
import functools

import jax
import jax.numpy as jnp
from jax import lax
from jax.experimental import pallas as pl
from jax.experimental.pallas import tpu as pltpu
from jax.experimental.pallas import tpu_sc as plsc

NUM_WORKERS = 32
LANES = 16
CHUNK = 128


@functools.lru_cache(maxsize=None)
def _build(V, D, N):
    assert D % LANES == 0
    per_w = N // NUM_WORKERS
    assert per_w * NUM_WORKERS == N and per_w % CHUNK == 0
    n_ch = per_w // CHUNK
    dsub = D // LANES
    mesh = plsc.VectorSubcoreMesh(core_axis_name="c", subcore_axis_name="s")

    @functools.partial(
        pl.kernel,
        mesh=mesh,
        out_type=jax.ShapeDtypeStruct((NUM_WORKERS, 3 * LANES), jnp.float32),
        scratch_types=[
            pltpu.VMEM((per_w,), jnp.int32),
            pltpu.VMEM((per_w,), jnp.int32),
            pltpu.VMEM((CHUNK, D), jnp.float32),
            pltpu.VMEM((CHUNK, D), jnp.float32),
            pltpu.VMEM((3 * LANES,), jnp.float32),
            pltpu.SemaphoreType.DMA,
            pltpu.SemaphoreType.DMA,
        ],
    )
    def phase1(table_h, left_h, right_h, out_h, idxl, idxr, lbuf, rbuf, stage,
               seml, semr):
        wid = lax.axis_index("s") * 2 + lax.axis_index("c")
        base = pl.multiple_of(wid * per_w, 8)
        pltpu.sync_copy(left_h.at[pl.ds(base, per_w)], idxl)
        pltpu.sync_copy(right_h.at[pl.ds(base, per_w)], idxr)

        def chunk_body(g, accs):
            off = pl.multiple_of(g * CHUNK, 8)
            cl = pltpu.async_copy(table_h.at[idxl.at[pl.ds(off, CHUNK)]],
                                  lbuf, seml)
            cr = pltpu.async_copy(table_h.at[idxr.at[pl.ds(off, CHUNK)]],
                                  rbuf, semr)
            cl.wait()
            cr.wait()

            def row_body(r, a):
                a = list(a)
                for j in range(dsub):
                    lv = lbuf[r, pl.ds(j * LANES, LANES)]
                    rv = rbuf[r, pl.ds(j * LANES, LANES)]
                    a[j] = a[j] + lv * rv
                    a[dsub + j] = a[dsub + j] + lv * lv
                    a[2 * dsub + j] = a[2 * dsub + j] + rv * rv
                return tuple(a)

            return lax.fori_loop(0, CHUNK, row_body, accs)

        zero = jnp.zeros((LANES,), jnp.float32)
        accs = lax.fori_loop(0, n_ch, chunk_body,
                             tuple(zero for _ in range(3 * dsub)))

        def tree_sum(vs):
            vs = list(vs)
            while len(vs) > 1:
                vs = [vs[i] + vs[i + 1] for i in range(0, len(vs) - 1, 2)] + (
                    [vs[-1]] if len(vs) % 2 else [])
            return vs[0]

        stage[pl.ds(0, LANES)] = tree_sum(accs[0:dsub])
        stage[pl.ds(LANES, LANES)] = tree_sum(accs[dsub:2 * dsub])
        stage[pl.ds(2 * LANES, LANES)] = tree_sum(accs[2 * dsub:3 * dsub])
        pltpu.sync_copy(stage, out_h.at[wid])

    @functools.partial(
        pl.kernel,
        mesh=mesh,
        out_type=jax.ShapeDtypeStruct((N, D), jnp.float32),
        scratch_types=[
            pltpu.VMEM((per_w,), jnp.int32),
            pltpu.VMEM((CHUNK, D), jnp.float32),
            pltpu.VMEM((LANES,), jnp.float32),
            pltpu.SemaphoreType.DMA,
        ],
    )
    def phase2(table_h, right_h, scale_h, out_h, idxr, rbuf, sv, sem):
        wid = lax.axis_index("s") * 2 + lax.axis_index("c")
        base = pl.multiple_of(wid * per_w, 8)
        pltpu.sync_copy(right_h.at[pl.ds(base, per_w)], idxr)
        pltpu.sync_copy(scale_h, sv)
        s = sv[...]

        def chunk_body(g, carry):
            off = pl.multiple_of(g * CHUNK, 8)
            pltpu.async_copy(table_h.at[idxr.at[pl.ds(off, CHUNK)]],
                             rbuf, sem).wait()

            def row_body(r, c):
                for j in range(dsub):
                    rbuf[r, pl.ds(j * LANES, LANES)] = (
                        rbuf[r, pl.ds(j * LANES, LANES)] * s)
                return c

            lax.fori_loop(0, CHUNK, row_body, 0)
            pltpu.sync_copy(rbuf, out_h.at[pl.ds(base + off, CHUNK)])
            return carry

        lax.fori_loop(0, n_ch, chunk_body, 0)

    return phase1, phase2


def kernel(table, left, right):
    V, D = table.shape
    B, H = left.shape
    N = B * H
    phase1, phase2 = _build(V, D, N)
    li = left.reshape(-1).astype(jnp.int32)
    ri = right.reshape(-1).astype(jnp.int32)
    part = phase1(table, li, ri)
    dot = jnp.sum(part[:, 0:LANES])
    ssq_l = jnp.sum(part[:, LANES:2 * LANES])
    ssq_r = jnp.sum(part[:, 2 * LANES:3 * LANES])
    fro_l = jnp.sqrt(ssq_l)
    fro_r = jnp.sqrt(ssq_r)
    loss = dot / (fro_l * fro_r)
    scale = jnp.full((LANES,), 1.0, jnp.float32) / fro_r
    out = phase2(table, ri, scale.astype(jnp.float32))
    return out.reshape(B, H, D), -loss


# speedup vs baseline: 4.7637x; 1.3593x over previous
import functools

import jax
import jax.numpy as jnp
from jax import lax
from jax.experimental import pallas as pl
from jax.experimental.pallas import tpu as pltpu
from jax.experimental.pallas import tpu_sc as plsc

NUM_WORKERS = 32
LANES = 16
CHUNK = 128


@functools.lru_cache(maxsize=None)
def _build_phase1(V, D, N):
    assert D % LANES == 0
    per_w = N // NUM_WORKERS
    assert per_w * NUM_WORKERS == N and per_w % (2 * CHUNK) == 0
    n_pairs = per_w // (2 * CHUNK)
    dsub = D // LANES
    mesh = plsc.VectorSubcoreMesh(core_axis_name="c", subcore_axis_name="s")

    @functools.partial(
        pl.kernel,
        mesh=mesh,
        out_type=(
            jax.ShapeDtypeStruct((NUM_WORKERS, 3 * LANES), jnp.float32),
            jax.ShapeDtypeStruct((N, D), jnp.float32),
        ),
        scratch_types=[
            pltpu.VMEM((per_w,), jnp.int32),
            pltpu.VMEM((per_w,), jnp.int32),
            pltpu.VMEM((CHUNK, D), jnp.float32),
            pltpu.VMEM((CHUNK, D), jnp.float32),
            pltpu.VMEM((CHUNK, D), jnp.float32),
            pltpu.VMEM((CHUNK, D), jnp.float32),
            pltpu.VMEM((3 * LANES,), jnp.float32),
            pltpu.SemaphoreType.DMA,
            pltpu.SemaphoreType.DMA,
            pltpu.SemaphoreType.DMA,
            pltpu.SemaphoreType.DMA,
            pltpu.SemaphoreType.DMA,
            pltpu.SemaphoreType.DMA,
        ],
    )
    def phase1(table_h, left_h, right_h, part_h, raw_h,
               idxl, idxr, lb0, lb1, rb0, rb1, stage,
               sgl0, sgl1, sgr0, sgr1, sw0, sw1):
        wid = lax.axis_index("s") * 2 + lax.axis_index("c")
        base = pl.multiple_of(wid * per_w, 8)
        pltpu.sync_copy(left_h.at[pl.ds(base, per_w)], idxl)
        pltpu.sync_copy(right_h.at[pl.ds(base, per_w)], idxr)
        lb, rb = (lb0, lb1), (rb0, rb1)
        sgl, sgr, sw = (sgl0, sgl1), (sgr0, sgr1), (sw0, sw1)

        def fire_gather(c, b):
            off = pl.multiple_of(c * CHUNK, 8)
            pltpu.async_copy(table_h.at[idxl.at[pl.ds(off, CHUNK)]],
                             lb[b], sgl[b])
            pltpu.async_copy(table_h.at[idxr.at[pl.ds(off, CHUNK)]],
                             rb[b], sgr[b])

        def wait_gather(b):
            pltpu.make_async_copy(table_h.at[pl.ds(0, CHUNK)],
                                  lb[b], sgl[b]).wait()
            pltpu.make_async_copy(table_h.at[pl.ds(0, CHUNK)],
                                  rb[b], sgr[b]).wait()

        def fire_write(c, b):
            off = pl.multiple_of(base + c * CHUNK, 8)
            pltpu.async_copy(rb[b], raw_h.at[pl.ds(off, CHUNK)], sw[b])

        def wait_write(b):
            pltpu.make_async_copy(rb[b], raw_h.at[pl.ds(0, CHUNK)],
                                  sw[b]).wait()

        def compute(b, accs):
            def row_body(r, a):
                a = list(a)
                for j in range(dsub):
                    lv = lb[b][r, pl.ds(j * LANES, LANES)]
                    rv = rb[b][r, pl.ds(j * LANES, LANES)]
                    a[j] = a[j] + lv * rv
                    a[dsub + j] = a[dsub + j] + lv * lv
                    a[2 * dsub + j] = a[2 * dsub + j] + rv * rv
                return tuple(a)
            return lax.fori_loop(0, CHUNK, row_body, accs)

        fire_gather(0, 0)

        def pair_body(g, accs):
            @pl.when(g > 0)
            def _():
                wait_write(1)
            fire_gather(2 * g + 1, 1)
            wait_gather(0)
            accs = compute(0, accs)
            fire_write(2 * g, 0)
            wait_write(0)
            @pl.when(g < n_pairs - 1)
            def _():
                fire_gather(2 * g + 2, 0)
            wait_gather(1)
            accs = compute(1, accs)
            fire_write(2 * g + 1, 1)
            return accs

        zero = jnp.zeros((LANES,), jnp.float32)
        accs = lax.fori_loop(0, n_pairs, pair_body,
                             tuple(zero for _ in range(3 * dsub)))
        wait_write(1)

        def tree_sum(vs):
            vs = list(vs)
            while len(vs) > 1:
                vs = [vs[i] + vs[i + 1] for i in range(0, len(vs) - 1, 2)] + (
                    [vs[-1]] if len(vs) % 2 else [])
            return vs[0]

        stage[pl.ds(0, LANES)] = tree_sum(accs[0:dsub])
        stage[pl.ds(LANES, LANES)] = tree_sum(accs[dsub:2 * dsub])
        stage[pl.ds(2 * LANES, LANES)] = tree_sum(accs[2 * dsub:3 * dsub])
        pltpu.sync_copy(stage, part_h.at[wid])

    return phase1


@functools.lru_cache(maxsize=None)
def _build_phase2(B, H, D):
    BB = 16
    assert B % BB == 0

    def scale_fn(scale_ref, raw_ref, out_ref):
        s = scale_ref[0]
        for k in range(BB):
            out_ref[k, :, :] = raw_ref[pl.ds(k * H, H), :] * s

    return pl.pallas_call(
        scale_fn,
        grid=(B // BB,),
        in_specs=[
            pl.BlockSpec(memory_space=pltpu.SMEM),
            pl.BlockSpec((BB * H, D), lambda i: (i, 0)),
        ],
        out_specs=pl.BlockSpec((BB, H, D), lambda i: (i, 0, 0)),
        out_shape=jax.ShapeDtypeStruct((B, H, D), jnp.float32),
    )


def kernel(table, left, right):
    V, D = table.shape
    B, H = left.shape
    N = B * H
    phase1 = _build_phase1(V, D, N)
    phase2 = _build_phase2(B, H, D)
    li = left.reshape(-1).astype(jnp.int32)
    ri = right.reshape(-1).astype(jnp.int32)
    part, raw = phase1(table, li, ri)
    dot = jnp.sum(part[:, 0:LANES])
    ssq_l = jnp.sum(part[:, LANES:2 * LANES])
    ssq_r = jnp.sum(part[:, 2 * LANES:3 * LANES])
    fro_l = jnp.sqrt(ssq_l)
    fro_r = jnp.sqrt(ssq_r)
    loss = dot / (fro_l * fro_r)
    scale = (1.0 / fro_r).reshape(1).astype(jnp.float32)
    out = phase2(scale, raw)
    return out, -loss


# speedup vs baseline: 6.1120x; 1.2830x over previous
import functools

import jax
import jax.numpy as jnp
from jax import lax
from jax.experimental import pallas as pl
from jax.experimental.pallas import tpu as pltpu
from jax.experimental.pallas import tpu_sc as plsc

NUM_WORKERS = 32
LANES = 16
CHUNK = 128


@functools.lru_cache(maxsize=None)
def _build_phase1(V, D, N):
    assert D % LANES == 0
    per_w = N // NUM_WORKERS
    assert per_w * NUM_WORKERS == N and per_w % (2 * CHUNK) == 0
    n_pairs = per_w // (2 * CHUNK)
    dsub = D // LANES
    mesh = plsc.VectorSubcoreMesh(core_axis_name="c", subcore_axis_name="s")

    @functools.partial(
        pl.kernel,
        mesh=mesh,
        out_type=(
            jax.ShapeDtypeStruct((NUM_WORKERS, 3 * LANES), jnp.float32),
            jax.ShapeDtypeStruct((N, D), jnp.float32),
        ),
        scratch_types=[
            pltpu.VMEM((per_w,), jnp.int32),
            pltpu.VMEM((per_w,), jnp.int32),
            pltpu.VMEM((CHUNK, D), jnp.float32),
            pltpu.VMEM((CHUNK, D), jnp.float32),
            pltpu.VMEM((CHUNK, D), jnp.float32),
            pltpu.VMEM((CHUNK, D), jnp.float32),
            pltpu.VMEM((3 * LANES,), jnp.float32),
            pltpu.SemaphoreType.DMA,
            pltpu.SemaphoreType.DMA,
            pltpu.SemaphoreType.DMA,
            pltpu.SemaphoreType.DMA,
            pltpu.SemaphoreType.DMA,
            pltpu.SemaphoreType.DMA,
        ],
    )
    def phase1(table_h, left_h, right_h, part_h, raw_h,
               idxl, idxr, lb0, lb1, rb0, rb1, stage,
               sgl0, sgl1, sgr0, sgr1, sw0, sw1):
        wid = lax.axis_index("s") * 2 + lax.axis_index("c")
        base = pl.multiple_of(wid * per_w, 8)
        pltpu.sync_copy(left_h.at[pl.ds(base, per_w)], idxl)
        pltpu.sync_copy(right_h.at[pl.ds(base, per_w)], idxr)
        lb, rb = (lb0, lb1), (rb0, rb1)
        sgl, sgr, sw = (sgl0, sgl1), (sgr0, sgr1), (sw0, sw1)

        def fire_gather(c, b):
            off = pl.multiple_of(c * CHUNK, 8)
            pltpu.async_copy(table_h.at[idxl.at[pl.ds(off, CHUNK)]],
                             lb[b], sgl[b])
            pltpu.async_copy(table_h.at[idxr.at[pl.ds(off, CHUNK)]],
                             rb[b], sgr[b])

        def wait_gather(b):
            pltpu.make_async_copy(table_h.at[pl.ds(0, CHUNK)],
                                  lb[b], sgl[b]).wait()
            pltpu.make_async_copy(table_h.at[pl.ds(0, CHUNK)],
                                  rb[b], sgr[b]).wait()

        def fire_write(c, b):
            off = pl.multiple_of(base + c * CHUNK, 8)
            pltpu.async_copy(rb[b], raw_h.at[pl.ds(off, CHUNK)], sw[b])

        def wait_write(b):
            pltpu.make_async_copy(rb[b], raw_h.at[pl.ds(0, CHUNK)],
                                  sw[b]).wait()

        UNROLL = 4

        def compute(b, accs):
            def row_body(r4, a):
                a = list(a)
                for u in range(UNROLL):
                    r = r4 * UNROLL + u
                    for j in range(dsub):
                        lv = lb[b][r, pl.ds(j * LANES, LANES)]
                        rv = rb[b][r, pl.ds(j * LANES, LANES)]
                        a[j] = a[j] + lv * rv
                        a[dsub + j] = a[dsub + j] + lv * lv
                        a[2 * dsub + j] = a[2 * dsub + j] + rv * rv
                return tuple(a)
            return lax.fori_loop(0, CHUNK // UNROLL, row_body, accs)

        fire_gather(0, 0)

        def pair_body(g, accs):
            @pl.when(g > 0)
            def _():
                wait_write(1)
            fire_gather(2 * g + 1, 1)
            wait_gather(0)
            accs = compute(0, accs)
            fire_write(2 * g, 0)
            wait_write(0)
            @pl.when(g < n_pairs - 1)
            def _():
                fire_gather(2 * g + 2, 0)
            wait_gather(1)
            accs = compute(1, accs)
            fire_write(2 * g + 1, 1)
            return accs

        zero = jnp.zeros((LANES,), jnp.float32)
        accs = lax.fori_loop(0, n_pairs, pair_body,
                             tuple(zero for _ in range(3 * dsub)))
        wait_write(1)

        def tree_sum(vs):
            vs = list(vs)
            while len(vs) > 1:
                vs = [vs[i] + vs[i + 1] for i in range(0, len(vs) - 1, 2)] + (
                    [vs[-1]] if len(vs) % 2 else [])
            return vs[0]

        stage[pl.ds(0, LANES)] = tree_sum(accs[0:dsub])
        stage[pl.ds(LANES, LANES)] = tree_sum(accs[dsub:2 * dsub])
        stage[pl.ds(2 * LANES, LANES)] = tree_sum(accs[2 * dsub:3 * dsub])
        pltpu.sync_copy(stage, part_h.at[wid])

    return phase1


@functools.lru_cache(maxsize=None)
def _build_phase2(B, H, D):
    BB = 64
    assert B % BB == 0

    def scale_fn(scale_ref, raw_ref, out_ref):
        s = scale_ref[0]
        for k in range(BB):
            out_ref[k, :, :] = raw_ref[pl.ds(k * H, H), :] * s

    return pl.pallas_call(
        scale_fn,
        grid=(B // BB,),
        in_specs=[
            pl.BlockSpec(memory_space=pltpu.SMEM),
            pl.BlockSpec((BB * H, D), lambda i: (i, 0)),
        ],
        out_specs=pl.BlockSpec((BB, H, D), lambda i: (i, 0, 0)),
        out_shape=jax.ShapeDtypeStruct((B, H, D), jnp.float32),
    )


def kernel(table, left, right):
    V, D = table.shape
    B, H = left.shape
    N = B * H
    phase1 = _build_phase1(V, D, N)
    phase2 = _build_phase2(B, H, D)
    li = left.reshape(-1).astype(jnp.int32)
    ri = right.reshape(-1).astype(jnp.int32)
    part, raw = phase1(table, li, ri)
    dot = jnp.sum(part[:, 0:LANES])
    ssq_l = jnp.sum(part[:, LANES:2 * LANES])
    ssq_r = jnp.sum(part[:, 2 * LANES:3 * LANES])
    fro_l = jnp.sqrt(ssq_l)
    fro_r = jnp.sqrt(ssq_r)
    loss = dot / (fro_l * fro_r)
    scale = (1.0 / fro_r).reshape(1).astype(jnp.float32)
    out = phase2(scale, raw)
    return out, -loss


# speedup vs baseline: 8.2053x; 1.3425x over previous
import functools

import jax
import jax.numpy as jnp
from jax import lax
from jax.experimental import pallas as pl
from jax.experimental.pallas import tpu as pltpu
from jax.experimental.pallas import tpu_sc as plsc

NUM_WORKERS = 32
LANES = 16
CHUNK = 128


@functools.lru_cache(maxsize=None)
def _build_phase1(V, D, N):
    assert D % LANES == 0
    per_w = N // NUM_WORKERS
    assert per_w * NUM_WORKERS == N and per_w % (2 * CHUNK) == 0
    n_pairs = per_w // (2 * CHUNK)
    dsub = D // LANES
    mesh = plsc.VectorSubcoreMesh(core_axis_name="c", subcore_axis_name="s")

    @functools.partial(
        pl.kernel,
        mesh=mesh,
        out_type=(
            jax.ShapeDtypeStruct((NUM_WORKERS, 3 * LANES), jnp.float32),
            jax.ShapeDtypeStruct((N, D), jnp.float32),
        ),
        scratch_types=[
            pltpu.VMEM((per_w,), jnp.int32),
            pltpu.VMEM((per_w,), jnp.int32),
            pltpu.VMEM((CHUNK, D), jnp.float32),
            pltpu.VMEM((CHUNK, D), jnp.float32),
            pltpu.VMEM((CHUNK, D), jnp.float32),
            pltpu.VMEM((CHUNK, D), jnp.float32),
            pltpu.VMEM((3 * LANES,), jnp.float32),
            pltpu.SemaphoreType.DMA,
            pltpu.SemaphoreType.DMA,
            pltpu.SemaphoreType.DMA,
            pltpu.SemaphoreType.DMA,
            pltpu.SemaphoreType.DMA,
            pltpu.SemaphoreType.DMA,
        ],
    )
    def phase1(table_h, left_h, right_h, part_h, raw_h,
               idxl, idxr, lb0, lb1, rb0, rb1, stage,
               sgl0, sgl1, sgr0, sgr1, sw0, sw1):
        wid = lax.axis_index("s") * 2 + lax.axis_index("c")
        base = pl.multiple_of(wid * per_w, 8)
        pltpu.sync_copy(left_h.at[pl.ds(base, per_w)], idxl)
        pltpu.sync_copy(right_h.at[pl.ds(base, per_w)], idxr)
        lb, rb = (lb0, lb1), (rb0, rb1)
        sgl, sgr, sw = (sgl0, sgl1), (sgr0, sgr1), (sw0, sw1)

        def fire_gather(c, b):
            off = pl.multiple_of(c * CHUNK, 8)
            pltpu.async_copy(table_h.at[idxl.at[pl.ds(off, CHUNK)]],
                             lb[b], sgl[b])
            pltpu.async_copy(table_h.at[idxr.at[pl.ds(off, CHUNK)]],
                             rb[b], sgr[b])

        def wait_gather(b):
            pltpu.make_async_copy(table_h.at[pl.ds(0, CHUNK)],
                                  lb[b], sgl[b]).wait()
            pltpu.make_async_copy(table_h.at[pl.ds(0, CHUNK)],
                                  rb[b], sgr[b]).wait()

        def fire_write(c, b):
            off = pl.multiple_of(base + c * CHUNK, 8)
            pltpu.async_copy(rb[b], raw_h.at[pl.ds(off, CHUNK)], sw[b])

        def wait_write(b):
            pltpu.make_async_copy(rb[b], raw_h.at[pl.ds(0, CHUNK)],
                                  sw[b]).wait()

        UNROLL = 4

        def compute(b, accs):
            def row_body(r4, a):
                a = list(a)
                for u in range(UNROLL):
                    r = r4 * UNROLL + u
                    for j in range(dsub):
                        lv = lb[b][r, pl.ds(j * LANES, LANES)]
                        rv = rb[b][r, pl.ds(j * LANES, LANES)]
                        a[j] = a[j] + lv * rv
                        a[dsub + j] = a[dsub + j] + lv * lv
                        a[2 * dsub + j] = a[2 * dsub + j] + rv * rv
                return tuple(a)
            return lax.fori_loop(0, CHUNK // UNROLL, row_body, accs)

        fire_gather(0, 0)

        def pair_body(g, accs):
            @pl.when(g > 0)
            def _():
                wait_write(1)
            fire_gather(2 * g + 1, 1)
            wait_gather(0)
            accs = compute(0, accs)
            fire_write(2 * g, 0)
            wait_write(0)
            @pl.when(g < n_pairs - 1)
            def _():
                fire_gather(2 * g + 2, 0)
            wait_gather(1)
            accs = compute(1, accs)
            fire_write(2 * g + 1, 1)
            return accs

        zero = jnp.zeros((LANES,), jnp.float32)
        accs = lax.fori_loop(0, n_pairs, pair_body,
                             tuple(zero for _ in range(3 * dsub)))
        wait_write(1)

        def tree_sum(vs):
            vs = list(vs)
            while len(vs) > 1:
                vs = [vs[i] + vs[i + 1] for i in range(0, len(vs) - 1, 2)] + (
                    [vs[-1]] if len(vs) % 2 else [])
            return vs[0]

        stage[pl.ds(0, LANES)] = tree_sum(accs[0:dsub])
        stage[pl.ds(LANES, LANES)] = tree_sum(accs[dsub:2 * dsub])
        stage[pl.ds(2 * LANES, LANES)] = tree_sum(accs[2 * dsub:3 * dsub])
        pltpu.sync_copy(stage, part_h.at[wid])

    return phase1


@functools.lru_cache(maxsize=None)
def _build_phase2(B, H, D):
    BB = 64
    assert B % BB == 0

    def scale_fn(scale_ref, raw_ref, out_ref):
        out_ref[...] = raw_ref[...] * scale_ref[0]

    return pl.pallas_call(
        scale_fn,
        grid=(B // BB,),
        in_specs=[
            pl.BlockSpec(memory_space=pltpu.SMEM),
            pl.BlockSpec((H, BB, D), lambda i: (0, i, 0)),
        ],
        out_specs=pl.BlockSpec((H, BB, D), lambda i: (0, i, 0)),
        out_shape=jax.ShapeDtypeStruct((H, B, D), jnp.float32),
    )


def kernel(table, left, right):
    V, D = table.shape
    B, H = left.shape
    N = B * H
    phase1 = _build_phase1(V, D, N)
    phase2 = _build_phase2(B, H, D)
    li = left.astype(jnp.int32).T.reshape(-1)
    ri = right.astype(jnp.int32).T.reshape(-1)
    part, raw = phase1(table, li, ri)
    dot = jnp.sum(part[:, 0:LANES])
    ssq_l = jnp.sum(part[:, LANES:2 * LANES])
    ssq_r = jnp.sum(part[:, 2 * LANES:3 * LANES])
    fro_l = jnp.sqrt(ssq_l)
    fro_r = jnp.sqrt(ssq_r)
    loss = dot / (fro_l * fro_r)
    scale = (1.0 / fro_r).reshape(1).astype(jnp.float32)
    out_t = phase2(scale, raw.reshape(H, B, D))
    return out_t.transpose(1, 0, 2), -loss


# speedup vs baseline: 8.3824x; 1.0216x over previous
import functools

import jax
import jax.numpy as jnp
from jax import lax
from jax.experimental import pallas as pl
from jax.experimental.pallas import tpu as pltpu
from jax.experimental.pallas import tpu_sc as plsc

NUM_WORKERS = 32
LANES = 16
CHUNK = 128


@functools.lru_cache(maxsize=None)
def _build_phase1(V, D, N):
    assert D % LANES == 0
    per_w = N // NUM_WORKERS
    assert per_w * NUM_WORKERS == N and per_w % CHUNK == 0
    n_ch = per_w // CHUNK
    dsub = D // LANES
    mesh = plsc.VectorSubcoreMesh(core_axis_name="c", subcore_axis_name="s")

    @functools.partial(
        pl.kernel,
        mesh=mesh,
        out_type=(
            jax.ShapeDtypeStruct((NUM_WORKERS, 3 * LANES), jnp.float32),
            jax.ShapeDtypeStruct((N, D), jnp.float32),
        ),
        scratch_types=[
            pltpu.VMEM((per_w,), jnp.int32),
            pltpu.VMEM((per_w,), jnp.int32),
            pltpu.VMEM((CHUNK, D), jnp.float32),
            pltpu.VMEM((CHUNK, D), jnp.float32),
            pltpu.VMEM((CHUNK, D), jnp.float32),
            pltpu.VMEM((CHUNK, D), jnp.float32),
            pltpu.VMEM((3 * LANES,), jnp.float32),
            pltpu.SemaphoreType.DMA,
            pltpu.SemaphoreType.DMA,
            pltpu.SemaphoreType.DMA,
            pltpu.SemaphoreType.DMA,
            pltpu.SemaphoreType.DMA,
            pltpu.SemaphoreType.DMA,
        ],
    )
    def phase1(table_h, left_h, right_h, part_h, raw_h,
               idxl, idxr, lb0, lb1, rb0, rb1, stage,
               sgl0, sgl1, sgr0, sgr1, sw0, sw1):
        wid = lax.axis_index("s") * 2 + lax.axis_index("c")
        base = pl.multiple_of(wid * per_w, 8)
        pltpu.sync_copy(left_h.at[pl.ds(base, per_w)], idxl)
        pltpu.sync_copy(right_h.at[pl.ds(base, per_w)], idxr)
        lb, rb = (lb0, lb1), (rb0, rb1)
        sgl, sgr, sw = (sgl0, sgl1), (sgr0, sgr1), (sw0, sw1)

        def fire_gather(c, b):
            off = pl.multiple_of(c * CHUNK, 8)
            pltpu.async_copy(table_h.at[idxl.at[pl.ds(off, CHUNK)]],
                             lb[b], sgl[b])
            pltpu.async_copy(table_h.at[idxr.at[pl.ds(off, CHUNK)]],
                             rb[b], sgr[b])

        def wait_gather(b):
            pltpu.make_async_copy(table_h.at[pl.ds(0, CHUNK)],
                                  lb[b], sgl[b]).wait()
            pltpu.make_async_copy(table_h.at[pl.ds(0, CHUNK)],
                                  rb[b], sgr[b]).wait()

        def fire_write(c, b):
            off = pl.multiple_of(base + c * CHUNK, 8)
            pltpu.async_copy(rb[b], raw_h.at[pl.ds(off, CHUNK)], sw[b])

        def wait_write(b):
            pltpu.make_async_copy(rb[b], raw_h.at[pl.ds(0, CHUNK)],
                                  sw[b]).wait()

        def compute(b, accs):
            def row_body(r, a):
                a = list(a)
                for j in range(dsub):
                    lv = lb[b][r, pl.ds(j * LANES, LANES)]
                    rv = rb[b][r, pl.ds(j * LANES, LANES)]
                    a[j] = a[j] + lv * rv
                    a[dsub + j] = a[dsub + j] + lv * lv
                    a[2 * dsub + j] = a[2 * dsub + j] + rv * rv
                return tuple(a)
            return lax.fori_loop(0, CHUNK, row_body, accs)

        n_pairs = n_ch // 2
        fire_gather(0, 0)

        def pair_body(g, accs):
            @pl.when(g > 0)
            def _():
                wait_write(1)
            fire_gather(2 * g + 1, 1)
            wait_gather(0)
            accs = compute(0, accs)
            fire_write(2 * g, 0)
            wait_write(0)
            @pl.when(g < n_pairs - 1)
            def _():
                fire_gather(2 * g + 2, 0)
            wait_gather(1)
            accs = compute(1, accs)
            fire_write(2 * g + 1, 1)
            return accs

        zero = jnp.zeros((LANES,), jnp.float32)
        accs = lax.fori_loop(0, n_pairs, pair_body,
                             tuple(zero for _ in range(3 * dsub)))
        wait_write(1)

        def tree_sum(vs):
            vs = list(vs)
            while len(vs) > 1:
                vs = [vs[i] + vs[i + 1] for i in range(0, len(vs) - 1, 2)] + (
                    [vs[-1]] if len(vs) % 2 else [])
            return vs[0]

        stage[pl.ds(0, LANES)] = tree_sum(accs[0:dsub])
        stage[pl.ds(LANES, LANES)] = tree_sum(accs[dsub:2 * dsub])
        stage[pl.ds(2 * LANES, LANES)] = tree_sum(accs[2 * dsub:3 * dsub])
        pltpu.sync_copy(stage, part_h.at[wid])

    return phase1


@functools.lru_cache(maxsize=None)
def _build_phase2(B, H, D):
    BB = 64
    assert B % BB == 0

    def scale_fn(scale_ref, raw_ref, out_ref):
        out_ref[...] = raw_ref[...] * scale_ref[0]

    return pl.pallas_call(
        scale_fn,
        grid=(B // BB,),
        in_specs=[
            pl.BlockSpec(memory_space=pltpu.SMEM),
            pl.BlockSpec((H, BB, D), lambda i: (0, i, 0)),
        ],
        out_specs=pl.BlockSpec((H, BB, D), lambda i: (0, i, 0)),
        out_shape=jax.ShapeDtypeStruct((H, B, D), jnp.float32),
    )


def kernel(table, left, right):
    V, D = table.shape
    B, H = left.shape
    N = B * H
    phase1 = _build_phase1(V, D, N)
    phase2 = _build_phase2(B, H, D)
    li = left.astype(jnp.int32).T.reshape(-1)
    ri = right.astype(jnp.int32).T.reshape(-1)
    part, raw = phase1(table, li, ri)
    dot = jnp.sum(part[:, 0:LANES])
    ssq_l = jnp.sum(part[:, LANES:2 * LANES])
    ssq_r = jnp.sum(part[:, 2 * LANES:3 * LANES])
    fro_l = jnp.sqrt(ssq_l)
    fro_r = jnp.sqrt(ssq_r)
    loss = dot / (fro_l * fro_r)
    scale = (1.0 / fro_r).reshape(1).astype(jnp.float32)
    out_t = phase2(scale, raw.reshape(H, B, D))
    return out_t.transpose(1, 0, 2), -loss


# speedup vs baseline: 8.4958x; 1.0135x over previous
import functools

import jax
import jax.numpy as jnp
from jax import lax
from jax.experimental import pallas as pl
from jax.experimental.pallas import tpu as pltpu
from jax.experimental.pallas import tpu_sc as plsc

NUM_WORKERS = 32
LANES = 16
CHUNK = 128


@functools.lru_cache(maxsize=None)
def _build_phase1(V, D, N):
    assert D % LANES == 0
    per_w = N // NUM_WORKERS
    assert per_w * NUM_WORKERS == N and per_w % CHUNK == 0
    n_ch = per_w // CHUNK
    dsub = D // LANES
    mesh = plsc.VectorSubcoreMesh(core_axis_name="c", subcore_axis_name="s")

    @functools.partial(
        pl.kernel,
        mesh=mesh,
        out_type=(
            jax.ShapeDtypeStruct((NUM_WORKERS, 3 * LANES), jnp.float32),
            jax.ShapeDtypeStruct((N, D), jnp.float32),
        ),
        scratch_types=[
            pltpu.VMEM((per_w,), jnp.int32),
            pltpu.VMEM((per_w,), jnp.int32),
            pltpu.VMEM((CHUNK, D), jnp.float32),
            pltpu.VMEM((CHUNK, D), jnp.float32),
            pltpu.VMEM((CHUNK, D), jnp.float32),
            pltpu.VMEM((CHUNK, D), jnp.float32),
            pltpu.VMEM((3 * LANES,), jnp.float32),
            pltpu.SemaphoreType.DMA,
            pltpu.SemaphoreType.DMA,
            pltpu.SemaphoreType.DMA,
            pltpu.SemaphoreType.DMA,
            pltpu.SemaphoreType.DMA,
            pltpu.SemaphoreType.DMA,
        ],
    )
    def phase1(table_h, left_h, right_h, part_h, raw_h,
               idxl, idxr, lb0, lb1, rb0, rb1, stage,
               sgl0, sgl1, sgr0, sgr1, sw0, sw1):
        wid = lax.axis_index("s") * 2 + lax.axis_index("c")
        base = pl.multiple_of(wid * per_w, 8)
        pltpu.sync_copy(left_h.at[pl.ds(base, per_w)], idxl)
        pltpu.sync_copy(right_h.at[pl.ds(base, per_w)], idxr)
        lb, rb = (lb0, lb1), (rb0, rb1)
        sgl, sgr, sw = (sgl0, sgl1), (sgr0, sgr1), (sw0, sw1)

        def fire_gather(c, b):
            off = pl.multiple_of(c * CHUNK, 8)
            pltpu.async_copy(table_h.at[idxl.at[pl.ds(off, CHUNK)]],
                             lb[b], sgl[b])
            pltpu.async_copy(table_h.at[idxr.at[pl.ds(off, CHUNK)]],
                             rb[b], sgr[b])

        def wait_gather(b):
            pltpu.make_async_copy(table_h.at[pl.ds(0, CHUNK)],
                                  lb[b], sgl[b]).wait()
            pltpu.make_async_copy(table_h.at[pl.ds(0, CHUNK)],
                                  rb[b], sgr[b]).wait()

        def fire_write(c, b):
            off = pl.multiple_of(base + c * CHUNK, 8)
            pltpu.async_copy(rb[b], raw_h.at[pl.ds(off, CHUNK)], sw[b])

        def wait_write(b):
            pltpu.make_async_copy(rb[b], raw_h.at[pl.ds(0, CHUNK)],
                                  sw[b]).wait()

        def compute(b, accs):
            def row_body(r, a):
                a = list(a)
                for j in range(dsub):
                    lv = lb[b][r, pl.ds(j * LANES, LANES)]
                    rv = rb[b][r, pl.ds(j * LANES, LANES)]
                    a[j] = a[j] + lv * rv
                    a[dsub + j] = a[dsub + j] + lv * lv
                    a[2 * dsub + j] = a[2 * dsub + j] + rv * rv
                return tuple(a)
            return lax.fori_loop(0, CHUNK, row_body, accs)

        n_pairs = n_ch // 2
        fire_gather(0, 0)

        def pair_body(g, accs):
            @pl.when(g > 0)
            def _():
                wait_write(1)
            fire_gather(2 * g + 1, 1)
            wait_gather(0)
            fire_write(2 * g, 0)
            accs = compute(0, accs)
            wait_write(0)
            @pl.when(g < n_pairs - 1)
            def _():
                fire_gather(2 * g + 2, 0)
            wait_gather(1)
            fire_write(2 * g + 1, 1)
            accs = compute(1, accs)
            return accs

        zero = jnp.zeros((LANES,), jnp.float32)
        accs = lax.fori_loop(0, n_pairs, pair_body,
                             tuple(zero for _ in range(3 * dsub)))
        wait_write(1)

        def tree_sum(vs):
            vs = list(vs)
            while len(vs) > 1:
                vs = [vs[i] + vs[i + 1] for i in range(0, len(vs) - 1, 2)] + (
                    [vs[-1]] if len(vs) % 2 else [])
            return vs[0]

        stage[pl.ds(0, LANES)] = tree_sum(accs[0:dsub])
        stage[pl.ds(LANES, LANES)] = tree_sum(accs[dsub:2 * dsub])
        stage[pl.ds(2 * LANES, LANES)] = tree_sum(accs[2 * dsub:3 * dsub])
        pltpu.sync_copy(stage, part_h.at[wid])

    return phase1


@functools.lru_cache(maxsize=None)
def _build_phase2(B, H, D):
    BB = 64
    assert B % BB == 0

    def scale_fn(scale_ref, raw_ref, out_ref):
        out_ref[...] = raw_ref[...] * scale_ref[0]

    return pl.pallas_call(
        scale_fn,
        grid=(B // BB,),
        in_specs=[
            pl.BlockSpec(memory_space=pltpu.SMEM),
            pl.BlockSpec((H, BB, D), lambda i: (0, i, 0)),
        ],
        out_specs=pl.BlockSpec((H, BB, D), lambda i: (0, i, 0)),
        out_shape=jax.ShapeDtypeStruct((H, B, D), jnp.float32),
    )


def kernel(table, left, right):
    V, D = table.shape
    B, H = left.shape
    N = B * H
    phase1 = _build_phase1(V, D, N)
    phase2 = _build_phase2(B, H, D)
    li = left.astype(jnp.int32).T.reshape(-1)
    ri = right.astype(jnp.int32).T.reshape(-1)
    part, raw = phase1(table, li, ri)
    dot = jnp.sum(part[:, 0:LANES])
    ssq_l = jnp.sum(part[:, LANES:2 * LANES])
    ssq_r = jnp.sum(part[:, 2 * LANES:3 * LANES])
    fro_l = jnp.sqrt(ssq_l)
    fro_r = jnp.sqrt(ssq_r)
    loss = dot / (fro_l * fro_r)
    scale = (1.0 / fro_r).reshape(1).astype(jnp.float32)
    out_t = phase2(scale, raw.reshape(H, B, D))
    return out_t.transpose(1, 0, 2), -loss


# speedup vs baseline: 8.9467x; 1.0531x over previous
import functools

import jax
import jax.numpy as jnp
from jax import lax
from jax.experimental import pallas as pl
from jax.experimental.pallas import tpu as pltpu
from jax.experimental.pallas import tpu_sc as plsc

NUM_WORKERS = 32
LANES = 16
CHUNK = 128


@functools.lru_cache(maxsize=None)
def _build_phase1(V, D, N):
    assert D % LANES == 0
    per_w = N // NUM_WORKERS
    assert per_w * NUM_WORKERS == N and per_w % CHUNK == 0
    n_ch = per_w // CHUNK
    dsub = D // LANES
    mesh = plsc.VectorSubcoreMesh(core_axis_name="c", subcore_axis_name="s")

    @functools.partial(
        pl.kernel,
        mesh=mesh,
        out_type=(
            jax.ShapeDtypeStruct((NUM_WORKERS, 3 * LANES), jnp.float32),
            jax.ShapeDtypeStruct((N, D), jnp.float32),
        ),
        scratch_types=[
            pltpu.VMEM((per_w,), jnp.int32),
            pltpu.VMEM((per_w,), jnp.int32),
            pltpu.VMEM((CHUNK, D), jnp.float32),
            pltpu.VMEM((CHUNK, D), jnp.float32),
            pltpu.VMEM((CHUNK, D), jnp.float32),
            pltpu.VMEM((CHUNK, D), jnp.float32),
            pltpu.VMEM((3 * LANES,), jnp.float32),
            pltpu.SemaphoreType.DMA,
            pltpu.SemaphoreType.DMA,
            pltpu.SemaphoreType.DMA,
            pltpu.SemaphoreType.DMA,
            pltpu.SemaphoreType.DMA,
            pltpu.SemaphoreType.DMA,
        ],
    )
    def phase1(table_h, left_h, right_h, part_h, raw_h,
               idxl, idxr, lb0, lb1, rb0, rb1, stage,
               sgl0, sgl1, sgr0, sgr1, sw0, sw1):
        wid = lax.axis_index("s") * 2 + lax.axis_index("c")
        base = pl.multiple_of(wid * per_w, 8)
        pltpu.sync_copy(left_h.at[pl.ds(base, per_w)], idxl)
        pltpu.sync_copy(right_h.at[pl.ds(base, per_w)], idxr)
        lb, rb = (lb0, lb1), (rb0, rb1)
        sgl, sgr, sw = (sgl0, sgl1), (sgr0, sgr1), (sw0, sw1)

        def fire_gather(c, b):
            off = pl.multiple_of(c * CHUNK, 8)
            pltpu.async_copy(table_h.at[idxl.at[pl.ds(off, CHUNK)]],
                             lb[b], sgl[b])
            pltpu.async_copy(table_h.at[idxr.at[pl.ds(off, CHUNK)]],
                             rb[b], sgr[b])

        def wait_gather(b):
            pltpu.make_async_copy(table_h.at[pl.ds(0, CHUNK)],
                                  lb[b], sgl[b]).wait()
            pltpu.make_async_copy(table_h.at[pl.ds(0, CHUNK)],
                                  rb[b], sgr[b]).wait()

        def fire_write(c, b):
            off = pl.multiple_of(base + c * CHUNK, 8)
            pltpu.async_copy(rb[b], raw_h.at[pl.ds(off, CHUNK)], sw[b])

        def wait_write(b):
            pltpu.make_async_copy(rb[b], raw_h.at[pl.ds(0, CHUNK)],
                                  sw[b]).wait()

        def compute(b, accs):
            def row_body(r, a):
                a = list(a)
                for j in range(dsub):
                    lv = lb[b][r, pl.ds(j * LANES, LANES)]
                    rv = rb[b][r, pl.ds(j * LANES, LANES)]
                    a[j] = a[j] + lv * rv
                    a[dsub + j] = a[dsub + j] + lv * lv
                    a[2 * dsub + j] = a[2 * dsub + j] + rv * rv
                return tuple(a)
            return lax.fori_loop(0, CHUNK, row_body, accs)

        n_pairs = n_ch // 2
        fire_gather(0, 0)

        def pair_body(g, accs):
            @pl.when(g > 0)
            def _():
                wait_write(1)
            fire_gather(2 * g + 1, 1)
            wait_gather(0)
            fire_write(2 * g, 0)
            accs = compute(0, accs)
            wait_write(0)
            @pl.when(g < n_pairs - 1)
            def _():
                fire_gather(2 * g + 2, 0)
            wait_gather(1)
            fire_write(2 * g + 1, 1)
            accs = compute(1, accs)
            return accs

        zero = jnp.zeros((LANES,), jnp.float32)
        accs = lax.fori_loop(0, n_pairs, pair_body,
                             tuple(zero for _ in range(3 * dsub)))
        wait_write(1)

        def tree_sum(vs):
            vs = list(vs)
            while len(vs) > 1:
                vs = [vs[i] + vs[i + 1] for i in range(0, len(vs) - 1, 2)] + (
                    [vs[-1]] if len(vs) % 2 else [])
            return vs[0]

        stage[pl.ds(0, LANES)] = tree_sum(accs[0:dsub])
        stage[pl.ds(LANES, LANES)] = tree_sum(accs[dsub:2 * dsub])
        stage[pl.ds(2 * LANES, LANES)] = tree_sum(accs[2 * dsub:3 * dsub])
        pltpu.sync_copy(stage, part_h.at[wid])

    return phase1


@functools.lru_cache(maxsize=None)
def _build_phase2(B, H, D):
    BB = 128
    assert B % BB == 0

    def scale_fn(scale_ref, raw_ref, out_ref):
        out_ref[...] = raw_ref[...] * scale_ref[0]

    return pl.pallas_call(
        scale_fn,
        grid=(B // BB,),
        in_specs=[
            pl.BlockSpec(memory_space=pltpu.SMEM),
            pl.BlockSpec((H, BB, D), lambda i: (0, i, 0)),
        ],
        out_specs=pl.BlockSpec((H, BB, D), lambda i: (0, i, 0)),
        out_shape=jax.ShapeDtypeStruct((H, B, D), jnp.float32),
    )


def kernel(table, left, right):
    V, D = table.shape
    B, H = left.shape
    N = B * H
    phase1 = _build_phase1(V, D, N)
    phase2 = _build_phase2(B, H, D)
    li = left.astype(jnp.int32).T.reshape(-1)
    ri = right.astype(jnp.int32).T.reshape(-1)
    part, raw = phase1(table, li, ri)
    dot = jnp.sum(part[:, 0:LANES])
    ssq_l = jnp.sum(part[:, LANES:2 * LANES])
    ssq_r = jnp.sum(part[:, 2 * LANES:3 * LANES])
    fro_l = jnp.sqrt(ssq_l)
    fro_r = jnp.sqrt(ssq_r)
    loss = dot / (fro_l * fro_r)
    scale = (1.0 / fro_r).reshape(1).astype(jnp.float32)
    out_t = phase2(scale, raw.reshape(H, B, D))
    return out_t.transpose(1, 0, 2), -loss


# speedup vs baseline: 9.0175x; 1.0079x over previous
import functools

import jax
import jax.numpy as jnp
from jax import lax
from jax.experimental import pallas as pl
from jax.experimental.pallas import tpu as pltpu
from jax.experimental.pallas import tpu_sc as plsc

NUM_WORKERS = 32
LANES = 16
CHUNK = 128


@functools.lru_cache(maxsize=None)
def _build_phase1(V, D, N):
    assert D % LANES == 0
    per_w = N // NUM_WORKERS
    assert per_w * NUM_WORKERS == N and per_w % CHUNK == 0
    n_ch = per_w // CHUNK
    dsub = D // LANES
    mesh = plsc.VectorSubcoreMesh(core_axis_name="c", subcore_axis_name="s")

    @functools.partial(
        pl.kernel,
        mesh=mesh,
        out_type=(
            jax.ShapeDtypeStruct((NUM_WORKERS, 3 * LANES), jnp.float32),
            jax.ShapeDtypeStruct((N, D), jnp.float32),
        ),
        scratch_types=[
            pltpu.VMEM((per_w,), jnp.int32),
            pltpu.VMEM((per_w,), jnp.int32),
            pltpu.VMEM((CHUNK, D), jnp.float32),
            pltpu.VMEM((CHUNK, D), jnp.float32),
            pltpu.VMEM((CHUNK, D), jnp.float32),
            pltpu.VMEM((CHUNK, D), jnp.float32),
            pltpu.VMEM((3 * LANES,), jnp.float32),
            pltpu.SemaphoreType.DMA,
            pltpu.SemaphoreType.DMA,
            pltpu.SemaphoreType.DMA,
            pltpu.SemaphoreType.DMA,
            pltpu.SemaphoreType.DMA,
            pltpu.SemaphoreType.DMA,
        ],
    )
    def phase1(table_h, left_h, right_h, part_h, raw_h,
               idxl, idxr, lb0, lb1, rb0, rb1, stage,
               sgl0, sgl1, sgr0, sgr1, sw0, sw1):
        wid = lax.axis_index("s") * 2 + lax.axis_index("c")
        base = pl.multiple_of(wid * per_w, 8)
        pltpu.sync_copy(left_h.at[pl.ds(base, per_w)], idxl)
        pltpu.sync_copy(right_h.at[pl.ds(base, per_w)], idxr)
        lb, rb = (lb0, lb1), (rb0, rb1)
        sgl, sgr, sw = (sgl0, sgl1), (sgr0, sgr1), (sw0, sw1)

        def fire_gather(c, b):
            off = pl.multiple_of(c * CHUNK, 8)
            pltpu.async_copy(table_h.at[idxl.at[pl.ds(off, CHUNK)]],
                             lb[b], sgl[b])
            pltpu.async_copy(table_h.at[idxr.at[pl.ds(off, CHUNK)]],
                             rb[b], sgr[b])

        def wait_gather(b):
            pltpu.make_async_copy(table_h.at[pl.ds(0, CHUNK)],
                                  lb[b], sgl[b]).wait()
            pltpu.make_async_copy(table_h.at[pl.ds(0, CHUNK)],
                                  rb[b], sgr[b]).wait()

        def fire_write(c, b):
            off = pl.multiple_of(base + c * CHUNK, 8)
            pltpu.async_copy(rb[b], raw_h.at[pl.ds(off, CHUNK)], sw[b])

        def wait_write(b):
            pltpu.make_async_copy(rb[b], raw_h.at[pl.ds(0, CHUNK)],
                                  sw[b]).wait()

        def compute(b, accs):
            def row_body(r, a):
                a = list(a)
                for j in range(dsub):
                    lv = lb[b][r, pl.ds(j * LANES, LANES)]
                    rv = rb[b][r, pl.ds(j * LANES, LANES)]
                    a[j] = a[j] + lv * rv
                    a[dsub + j] = a[dsub + j] + lv * lv
                    a[2 * dsub + j] = a[2 * dsub + j] + rv * rv
                return tuple(a)
            return lax.fori_loop(0, CHUNK, row_body, accs)

        n_pairs = n_ch // 2
        fire_gather(0, 0)

        def pair_body(g, accs):
            @pl.when(g > 0)
            def _():
                wait_write(1)
            fire_gather(2 * g + 1, 1)
            wait_gather(0)
            fire_write(2 * g, 0)
            accs = compute(0, accs)
            wait_write(0)
            @pl.when(g < n_pairs - 1)
            def _():
                fire_gather(2 * g + 2, 0)
            wait_gather(1)
            fire_write(2 * g + 1, 1)
            accs = compute(1, accs)
            return accs

        zero = jnp.zeros((LANES,), jnp.float32)
        accs = lax.fori_loop(0, n_pairs, pair_body,
                             tuple(zero for _ in range(3 * dsub)))
        wait_write(1)

        def tree_sum(vs):
            vs = list(vs)
            while len(vs) > 1:
                vs = [vs[i] + vs[i + 1] for i in range(0, len(vs) - 1, 2)] + (
                    [vs[-1]] if len(vs) % 2 else [])
            return vs[0]

        stage[pl.ds(0, LANES)] = tree_sum(accs[0:dsub])
        stage[pl.ds(LANES, LANES)] = tree_sum(accs[dsub:2 * dsub])
        stage[pl.ds(2 * LANES, LANES)] = tree_sum(accs[2 * dsub:3 * dsub])
        pltpu.sync_copy(stage, part_h.at[wid])

    return phase1


@functools.lru_cache(maxsize=None)
def _build_phase2(B, H, D):
    BB = 256
    assert B % BB == 0

    def scale_fn(scale_ref, raw_ref, out_ref):
        out_ref[...] = raw_ref[...] * scale_ref[0]

    return pl.pallas_call(
        scale_fn,
        grid=(B // BB,),
        in_specs=[
            pl.BlockSpec(memory_space=pltpu.SMEM),
            pl.BlockSpec((H, BB, D), lambda i: (0, i, 0)),
        ],
        out_specs=pl.BlockSpec((H, BB, D), lambda i: (0, i, 0)),
        out_shape=jax.ShapeDtypeStruct((H, B, D), jnp.float32),
    )


def kernel(table, left, right):
    V, D = table.shape
    B, H = left.shape
    N = B * H
    phase1 = _build_phase1(V, D, N)
    phase2 = _build_phase2(B, H, D)
    li = left.astype(jnp.int32).T.reshape(-1)
    ri = right.astype(jnp.int32).T.reshape(-1)
    part, raw = phase1(table, li, ri)
    dot = jnp.sum(part[:, 0:LANES])
    ssq_l = jnp.sum(part[:, LANES:2 * LANES])
    ssq_r = jnp.sum(part[:, 2 * LANES:3 * LANES])
    fro_l = jnp.sqrt(ssq_l)
    fro_r = jnp.sqrt(ssq_r)
    loss = dot / (fro_l * fro_r)
    scale = (1.0 / fro_r).reshape(1).astype(jnp.float32)
    out_t = phase2(scale, raw.reshape(H, B, D))
    return out_t.transpose(1, 0, 2), -loss
